# Initial kernel scaffold; baseline (speedup 1.0000x reference)
import jax
import jax.numpy as jnp
from jax.experimental import pallas as pl


def kernel(graph_x, edge_index, batch, W_gat, att_src, att_dst, gat_bias, bn_weight, bn_bias, head_W, head_b):
    raise NotImplementedError("write your pallas kernel here")


# speedup vs baseline: 20.7307x; 20.7307x over previous
import jax
import jax.numpy as jnp
from jax import lax
from jax.experimental import pallas as pl
from jax.experimental.pallas import tpu as pltpu
from jax.experimental.pallas import tpu_sc as plsc

N = 10000
E = 320000
D = 128
HID = 128
OUT = 16
B = 64

NC = 2
NS = 16
NW = NC * NS

EE = E + N
K = 32
T = 10336
NCHUNK = T // K
EEP = NW * T
NAP = N + 16
NP = 10240
ROWS_PER_TILE = NP // NS


def _vgather(x, idx):
    dnums = lax.GatherDimensionNumbers(
        offset_dims=(), collapsed_slice_dims=(0,), start_index_map=(0,))
    return lax.gather(
        x, idx[:, None], dimension_numbers=dnums, slice_sizes=(1,),
        mode=lax.GatherScatterMode.PROMISE_IN_BOUNDS)


def _mm_body(x_ref, wt_ref, a2_ref, h_ref, aux_ref):
    h = jnp.dot(x_ref[...], wt_ref[...], preferred_element_type=jnp.float32)
    h_ref[...] = h
    aux_ref[...] = jnp.dot(h, a2_ref[...], preferred_element_type=jnp.float32)


def _stage1(x, WT, A2):
    R = 1000
    grid = N // R
    return pl.pallas_call(
        _mm_body,
        grid=(grid,),
        in_specs=[
            pl.BlockSpec((R, D), lambda i: (i, 0)),
            pl.BlockSpec((D, HID), lambda i: (0, 0)),
            pl.BlockSpec((HID, HID), lambda i: (0, 0)),
        ],
        out_specs=[
            pl.BlockSpec((R, HID), lambda i: (i, 0)),
            pl.BlockSpec((R, HID), lambda i: (i, 0)),
        ],
        out_shape=[
            jax.ShapeDtypeStruct((N, HID), jnp.float32),
            jax.ShapeDtypeStruct((N, HID), jnp.float32),
        ],
    )(x, WT, A2)


def _sc_body(h_hbm, asrc_hbm, adst_hbm, ed_hbm,
             num_hbm, den_hbm,
             asrc_v, adst_v, ed_v, w_v, den_v, rows_v,
             num_sh, sem_ed, sem_g, sem_s):
    cid = lax.axis_index("c")
    sid = lax.axis_index("s")
    wid = sid * NC + cid

    pltpu.sync_copy(asrc_hbm, asrc_v)
    pltpu.sync_copy(adst_hbm, adst_v)

    zero16 = jnp.zeros((16,), jnp.float32)
    for r in range(2 * K):
        for q in range(8):
            rows_v[r, pl.ds(q * 16, 16)] = zero16

    def zero_den(c, carry):
        den_v[pl.ds(c * 16, 16)] = zero16
        return carry

    lax.fori_loop(0, NP // 16, zero_den, 0)

    base = sid * ROWS_PER_TILE

    def zero_num(c, carry):
        pltpu.sync_copy(rows_v, num_sh.at[pl.ds(base + c * 2 * K, 2 * K)])
        return carry

    lax.fori_loop(0, ROWS_PER_TILE // (2 * K), zero_num, 0)
    plsc.subcore_barrier()

    lanes = lax.broadcasted_iota(jnp.int32, (16,), 0)

    pltpu.async_copy(ed_hbm.at[wid, 0], ed_v.at[0], sem_ed)

    def chunk(j, carry):
        b = lax.rem(j, 2)
        nb = 1 - b
        boff = b * K

        pltpu.make_async_copy(ed_hbm.at[wid, j], ed_v.at[b], sem_ed).wait()
        @pl.when(j > 0)
        def _():
            pltpu.make_async_copy(
                rows_v.at[pl.ds(nb * K, K)],
                num_sh.at[ed_v.at[nb, 1]], sem_s).wait()

        @pl.when(j + 1 < NCHUNK)
        def _():
            pltpu.async_copy(ed_hbm.at[wid, j + 1], ed_v.at[nb], sem_ed)

        gcp = pltpu.async_copy(
            h_hbm.at[ed_v.at[b, 0]], rows_v.at[pl.ds(boff, K)], sem_g)

        for l in range(K // 16):
            s16 = ed_v[b, 0, pl.ds(l * 16, 16)]
            d16 = ed_v[b, 1, pl.ds(l * 16, 16)]
            a_s = plsc.load_gather(asrc_v, [s16])
            a_d = plsc.load_gather(adst_v, [d16])
            al = a_s + a_d
            al = jnp.where(al >= 0.0, al, al * 0.2)
            w = jnp.exp(al)
            w_v[pl.ds(boff + l * 16, 16)] = w
            k, v = plsc.sort_key_val(d16, w)
            cs = plsc.cumsum(v)
            prv = jnp.maximum(lanes - 1, 0)
            nxt = jnp.minimum(lanes + 1, 15)
            kprev = _vgather(k, prv)
            knext = _vgather(k, nxt)
            csprev = _vgather(cs, prv)
            is_first = (lanes == 0) | (k != kprev)
            is_last = (lanes == 15) | (k != knext)
            neg = jnp.where(lanes == 0, 0.0, -csprev)
            plsc.addupdate_scatter(den_v, [k], cs, mask=is_last)
            plsc.addupdate_scatter(den_v, [k], neg, mask=is_first)

        gcp.wait()
        for e in range(K):
            idx16 = jnp.broadcast_to(boff + e, (16,)).astype(jnp.int32)
            wsp = plsc.load_gather(w_v, [idx16])
            for q in range(8):
                rows_v[boff + e, pl.ds(q * 16, 16)] = (
                    rows_v[boff + e, pl.ds(q * 16, 16)] * wsp)
        pltpu.async_copy(
            rows_v.at[pl.ds(boff, K)], num_sh.at[ed_v.at[b, 1]],
            sem_s, add=True)
        return carry

    lax.fori_loop(0, NCHUNK, chunk, 0)
    lb = (NCHUNK - 1) % 2
    pltpu.make_async_copy(
        rows_v.at[pl.ds(lb * K, K)], num_sh.at[ed_v.at[lb, 1]],
        sem_s).wait()
    plsc.subcore_barrier()

    def wb_chunk(c, carry):
        off = base + c * 2 * K
        pltpu.sync_copy(num_sh.at[pl.ds(off, 2 * K)], rows_v)
        pltpu.sync_copy(rows_v, num_hbm.at[cid, pl.ds(off, 2 * K)])
        return carry

    lax.fori_loop(0, ROWS_PER_TILE // (2 * K), wb_chunk, 0)
    pltpu.sync_copy(den_v, den_hbm.at[wid])


def _stage2(h, asrc_p, adst_p, ed4):
    mesh = plsc.VectorSubcoreMesh(core_axis_name="c", subcore_axis_name="s")
    run = pl.kernel(
        _sc_body,
        out_type=[
            jax.ShapeDtypeStruct((NC, NP, HID), jnp.float32),
            jax.ShapeDtypeStruct((NW, NP), jnp.float32),
        ],
        mesh=mesh,
        scratch_types=[
            pltpu.VMEM((NAP,), jnp.float32),
            pltpu.VMEM((NAP,), jnp.float32),
            pltpu.VMEM((2, 2, K), jnp.int32),
            pltpu.VMEM((2 * K,), jnp.float32),
            pltpu.VMEM((NP,), jnp.float32),
            pltpu.VMEM((2 * K, HID), jnp.float32),
            pltpu.VMEM_SHARED((NP, HID), jnp.float32),
            pltpu.SemaphoreType.DMA,
            pltpu.SemaphoreType.DMA,
            pltpu.SemaphoreType.DMA,
        ],
        compiler_params=pltpu.CompilerParams(needs_layout_passes=False),
    )
    return run(h, asrc_p, adst_p, ed4)


def _fin_body(num_ref, den_ref, batch_ref, gb_ref, bnw_ref, bnb_ref,
              hwt_ref, hb_ref, out_ref, acc_ref, cnt_ref):
    i = pl.program_id(0)
    ni = pl.num_programs(0)

    @pl.when(i == 0)
    def _():
        acc_ref[...] = jnp.zeros_like(acc_ref)
        cnt_ref[...] = jnp.zeros_like(cnt_ref)

    num = num_ref[0] + num_ref[1]
    den = jnp.sum(den_ref[...], axis=0)
    rec = 1.0 / (den + 1e-16)
    o = num * rec[:, None]
    o = o + gb_ref[...]
    o = o * bnw_ref[...] + bnb_ref[...]
    o = jnp.maximum(o, 0.0)

    b = batch_ref[0, 0, :]
    rows = b.shape[0]
    gids = lax.broadcasted_iota(jnp.int32, (B, rows), 0)
    m = (gids == b[None, :]).astype(jnp.float32)
    acc_ref[...] += jnp.dot(m, o, preferred_element_type=jnp.float32)
    cnt_ref[...] += jnp.broadcast_to(
        jnp.sum(m, axis=1)[:, None], cnt_ref.shape)

    @pl.when(i == ni - 1)
    def _():
        pooled = acc_ref[...] / jnp.maximum(cnt_ref[...], 1.0)
        out_ref[...] = (
            jnp.dot(pooled, hwt_ref[...], preferred_element_type=jnp.float32)
            + hb_ref[...])


def _stage3(num, den, batch3, gb, bnw, bnb, hwt, hb):
    R = 1024
    grid = NP // R
    return pl.pallas_call(
        _fin_body,
        grid=(grid,),
        in_specs=[
            pl.BlockSpec((NC, R, HID), lambda i: (0, i, 0)),
            pl.BlockSpec((NW, R), lambda i: (0, i)),
            pl.BlockSpec((1, 1, R), lambda i: (i, 0, 0)),
            pl.BlockSpec((1, HID), lambda i: (0, 0)),
            pl.BlockSpec((1, HID), lambda i: (0, 0)),
            pl.BlockSpec((1, HID), lambda i: (0, 0)),
            pl.BlockSpec((HID, OUT), lambda i: (0, 0)),
            pl.BlockSpec((1, OUT), lambda i: (0, 0)),
        ],
        out_specs=pl.BlockSpec((B, OUT), lambda i: (0, 0)),
        out_shape=jax.ShapeDtypeStruct((B, OUT), jnp.float32),
        scratch_shapes=[
            pltpu.VMEM((B, HID), jnp.float32),
            pltpu.VMEM((B, HID), jnp.float32),
        ],
    )(num, den, batch3, gb, bnw, bnb, hwt, hb)


@jax.jit
def _run(graph_x, edge_index, batch, W_gat, att_src, att_dst, gat_bias,
         bn_weight, bn_bias, head_W, head_b):
    x = graph_x.astype(jnp.float32)

    A2 = jnp.zeros((HID, HID), jnp.float32)
    A2 = A2.at[:, 0].set(att_src).at[:, 1].set(att_dst)
    h, aux = _stage1(x, W_gat.T, A2)

    loop = jnp.arange(N, dtype=jnp.int32)
    src = jnp.concatenate(
        [edge_index[0], loop, jnp.zeros((EEP - EE,), jnp.int32)])
    dst = jnp.concatenate(
        [edge_index[1], loop, jnp.full((EEP - EE,), N, jnp.int32)])
    src3 = src.reshape(NW, NCHUNK, K)
    dst3 = dst.reshape(NW, NCHUNK, K)
    ed4 = jnp.stack([src3, dst3], axis=2)
    asrc_p = jnp.pad(aux[:, 0], (0, NAP - N))
    adst_p = jnp.pad(aux[:, 1], (0, NAP - N))

    num, den = _stage2(h, asrc_p, adst_p, ed4)

    bnw = (bn_weight / jnp.sqrt(1.0 + 1e-5)).reshape(1, HID)
    batch3 = jnp.pad(batch, (0, NP - N), constant_values=B).reshape(
        NP // 1024, 1, 1024)
    logits = _stage3(
        num, den, batch3, gat_bias.reshape(1, HID), bnw,
        bn_bias.reshape(1, HID), head_W.T, head_b.reshape(1, OUT))
    return logits


def kernel(graph_x, edge_index, batch, W_gat, att_src, att_dst, gat_bias,
           bn_weight, bn_bias, head_W, head_b):
    return _run(graph_x, edge_index, batch, W_gat, att_src, att_dst,
                gat_bias, bn_weight, bn_bias, head_W, head_b)


# speedup vs baseline: 32.9888x; 1.5913x over previous
import jax
import jax.numpy as jnp
from jax import lax
from jax.experimental import pallas as pl
from jax.experimental.pallas import tpu as pltpu
from jax.experimental.pallas import tpu_sc as plsc

N = 10000
E = 320000
D = 128
HID = 128
OUT = 16
B = 64

NC = 2
NS = 16
NW = NC * NS

EE = E + N
K = 32
T = 10336
NCHUNK = T // K
EEP = NW * T
NAP = N + 16
NP = 10240
ROWS_PER_TILE = NP // NS


def _vgather(x, idx):
    dnums = lax.GatherDimensionNumbers(
        offset_dims=(), collapsed_slice_dims=(0,), start_index_map=(0,))
    return lax.gather(
        x, idx[:, None], dimension_numbers=dnums, slice_sizes=(1,),
        mode=lax.GatherScatterMode.PROMISE_IN_BOUNDS)


def _mm_body(x_ref, wt_ref, a2_ref, h_ref, aux_ref):
    h = jnp.dot(x_ref[...], wt_ref[...], preferred_element_type=jnp.float32)
    h_ref[...] = h
    aux_ref[...] = jnp.dot(h, a2_ref[...], preferred_element_type=jnp.float32)


def _stage1(x, WT, A2):
    R = 1000
    grid = N // R
    return pl.pallas_call(
        _mm_body,
        grid=(grid,),
        in_specs=[
            pl.BlockSpec((R, D), lambda i: (i, 0)),
            pl.BlockSpec((D, HID), lambda i: (0, 0)),
            pl.BlockSpec((HID, HID), lambda i: (0, 0)),
        ],
        out_specs=[
            pl.BlockSpec((R, HID), lambda i: (i, 0)),
            pl.BlockSpec((R, HID), lambda i: (i, 0)),
        ],
        out_shape=[
            jax.ShapeDtypeStruct((N, HID), jnp.float32),
            jax.ShapeDtypeStruct((N, HID), jnp.float32),
        ],
    )(x, WT, A2)


def _sc_body(h_hbm, asrc_hbm, adst_hbm, ed_hbm,
             num_hbm, den_hbm,
             asrc_v, adst_v, ed_v, den_v, rows_v,
             num_sh, sem_e0, sem_e1, sem_e2, sem_g0, sem_g1, sem_s):
    cid = lax.axis_index("c")
    sid = lax.axis_index("s")
    wid = sid * NC + cid

    pltpu.sync_copy(asrc_hbm, asrc_v)
    pltpu.sync_copy(adst_hbm, adst_v)

    zero16 = jnp.zeros((16,), jnp.float32)
    for r in range(2 * K):
        for q in range(8):
            rows_v[r, pl.ds(q * 16, 16)] = zero16

    def zero_den(c, carry):
        den_v[pl.ds(c * 16, 16)] = zero16
        return carry

    lax.fori_loop(0, NP // 16, zero_den, 0)

    base = sid * ROWS_PER_TILE

    def zero_num(c, carry):
        pltpu.sync_copy(rows_v, num_sh.at[pl.ds(base + c * 2 * K, 2 * K)])
        return carry

    lax.fori_loop(0, ROWS_PER_TILE // (2 * K), zero_num, 0)
    plsc.subcore_barrier()

    lanes = lax.broadcasted_iota(jnp.int32, (16,), 0)

    pltpu.async_copy(ed_hbm.at[wid, 0], ed_v.at[0], sem_e0)
    pltpu.async_copy(ed_hbm.at[wid, 1], ed_v.at[1], sem_e1)
    pltpu.make_async_copy(ed_hbm.at[wid, 0], ed_v.at[0], sem_e0).wait()
    pltpu.async_copy(
        h_hbm.at[ed_v.at[0, 0]], rows_v.at[pl.ds(0, K)], sem_g0)

    def chunk(j, carry):
        b = lax.rem(j, 2)
        nb = 1 - b
        boff = b * K
        sl = lax.rem(j, 3)
        sl1 = lax.rem(j + 1, 3)
        sl2 = lax.rem(j + 2, 3)
        sem_g_b = [sem_g0, sem_g1]
        sem_e_all = [sem_e0, sem_e1, sem_e2]

        @pl.when(j > 0)
        def _():
            pltpu.make_async_copy(
                rows_v.at[pl.ds(nb * K, K)],
                num_sh.at[ed_v.at[sl2, 1]], sem_s).wait()

        @pl.when(j + 1 < NCHUNK)
        def _():
            for s in range(3):
                @pl.when(sl1 == s)
                def _():
                    pltpu.make_async_copy(
                        ed_hbm.at[wid, j + 1], ed_v.at[sl1],
                        sem_e_all[s]).wait()
            for s in range(2):
                @pl.when(nb == s)
                def _():
                    pltpu.async_copy(
                        h_hbm.at[ed_v.at[sl1, 0]],
                        rows_v.at[pl.ds(nb * K, K)], sem_g_b[s])

        @pl.when(j + 2 < NCHUNK)
        def _():
            for s in range(3):
                @pl.when(sl2 == s)
                def _():
                    pltpu.async_copy(
                        ed_hbm.at[wid, j + 2], ed_v.at[sl2], sem_e_all[s])

        w16s = []
        for l in range(K // 16):
            s16 = ed_v[sl, 0, pl.ds(l * 16, 16)]
            d16 = ed_v[sl, 1, pl.ds(l * 16, 16)]
            a_s = plsc.load_gather(asrc_v, [s16])
            a_d = plsc.load_gather(adst_v, [d16])
            al = a_s + a_d
            al = jnp.where(al >= 0.0, al, al * 0.2)
            w = jnp.exp(al)
            w16s.append(w)
            k, v = plsc.sort_key_val(d16, w)
            cs = plsc.cumsum(v)
            prv = jnp.maximum(lanes - 1, 0)
            nxt = jnp.minimum(lanes + 1, 15)
            kprev = _vgather(k, prv)
            knext = _vgather(k, nxt)
            csprev = _vgather(cs, prv)
            is_first = (lanes == 0) | (k != kprev)
            is_last = (lanes == 15) | (k != knext)
            neg = jnp.where(lanes == 0, 0.0, -csprev)
            plsc.addupdate_scatter(den_v, [k], cs, mask=is_last)
            plsc.addupdate_scatter(den_v, [k], neg, mask=is_first)

        for s in range(2):
            @pl.when(b == s)
            def _():
                pltpu.make_async_copy(
                    h_hbm.at[ed_v.at[sl, 0]],
                    rows_v.at[pl.ds(boff, K)], sem_g_b[s]).wait()
        for l in range(K // 16):
            for e in range(16):
                wsp = _vgather(w16s[l], jnp.full((16,), e, jnp.int32))
                r = boff + l * 16 + e
                for q in range(8):
                    rows_v[r, pl.ds(q * 16, 16)] = (
                        rows_v[r, pl.ds(q * 16, 16)] * wsp)
        pltpu.async_copy(
            rows_v.at[pl.ds(boff, K)], num_sh.at[ed_v.at[sl, 1]],
            sem_s, add=True)
        return carry

    lax.fori_loop(0, NCHUNK, chunk, 0)
    lb = (NCHUNK - 1) % 2
    lsl = (NCHUNK - 1) % 3
    pltpu.make_async_copy(
        rows_v.at[pl.ds(lb * K, K)], num_sh.at[ed_v.at[lsl, 1]],
        sem_s).wait()
    plsc.subcore_barrier()

    def wb_chunk(c, carry):
        off = base + c * 2 * K
        pltpu.sync_copy(num_sh.at[pl.ds(off, 2 * K)], rows_v)
        pltpu.sync_copy(rows_v, num_hbm.at[cid, pl.ds(off, 2 * K)])
        return carry

    lax.fori_loop(0, ROWS_PER_TILE // (2 * K), wb_chunk, 0)
    pltpu.sync_copy(den_v, den_hbm.at[wid])


def _stage2(h, asrc_p, adst_p, ed4):
    mesh = plsc.VectorSubcoreMesh(core_axis_name="c", subcore_axis_name="s")
    run = pl.kernel(
        _sc_body,
        out_type=[
            jax.ShapeDtypeStruct((NC, NP, HID), jnp.float32),
            jax.ShapeDtypeStruct((NW, NP), jnp.float32),
        ],
        mesh=mesh,
        scratch_types=[
            pltpu.VMEM((NAP,), jnp.float32),
            pltpu.VMEM((NAP,), jnp.float32),
            pltpu.VMEM((3, 2, K), jnp.int32),
            pltpu.VMEM((NP,), jnp.float32),
            pltpu.VMEM((2 * K, HID), jnp.float32),
            pltpu.VMEM_SHARED((NP, HID), jnp.float32),
            pltpu.SemaphoreType.DMA,
            pltpu.SemaphoreType.DMA,
            pltpu.SemaphoreType.DMA,
            pltpu.SemaphoreType.DMA,
            pltpu.SemaphoreType.DMA,
            pltpu.SemaphoreType.DMA,
        ],
        compiler_params=pltpu.CompilerParams(needs_layout_passes=False),
    )
    return run(h, asrc_p, adst_p, ed4)


def _fin_body(num_ref, den_ref, batch_ref, gb_ref, bnw_ref, bnb_ref,
              hwt_ref, hb_ref, out_ref, acc_ref, cnt_ref):
    i = pl.program_id(0)
    ni = pl.num_programs(0)

    @pl.when(i == 0)
    def _():
        acc_ref[...] = jnp.zeros_like(acc_ref)
        cnt_ref[...] = jnp.zeros_like(cnt_ref)

    num = num_ref[0] + num_ref[1]
    den = jnp.sum(den_ref[...], axis=0)
    rec = 1.0 / (den + 1e-16)
    o = num * rec[:, None]
    o = o + gb_ref[...]
    o = o * bnw_ref[...] + bnb_ref[...]
    o = jnp.maximum(o, 0.0)

    b = batch_ref[0, 0, :]
    rows = b.shape[0]
    gids = lax.broadcasted_iota(jnp.int32, (B, rows), 0)
    m = (gids == b[None, :]).astype(jnp.float32)
    acc_ref[...] += jnp.dot(m, o, preferred_element_type=jnp.float32)
    cnt_ref[...] += jnp.broadcast_to(
        jnp.sum(m, axis=1)[:, None], cnt_ref.shape)

    @pl.when(i == ni - 1)
    def _():
        pooled = acc_ref[...] / jnp.maximum(cnt_ref[...], 1.0)
        out_ref[...] = (
            jnp.dot(pooled, hwt_ref[...], preferred_element_type=jnp.float32)
            + hb_ref[...])


def _stage3(num, den, batch3, gb, bnw, bnb, hwt, hb):
    R = 1024
    grid = NP // R
    return pl.pallas_call(
        _fin_body,
        grid=(grid,),
        in_specs=[
            pl.BlockSpec((NC, R, HID), lambda i: (0, i, 0)),
            pl.BlockSpec((NW, R), lambda i: (0, i)),
            pl.BlockSpec((1, 1, R), lambda i: (i, 0, 0)),
            pl.BlockSpec((1, HID), lambda i: (0, 0)),
            pl.BlockSpec((1, HID), lambda i: (0, 0)),
            pl.BlockSpec((1, HID), lambda i: (0, 0)),
            pl.BlockSpec((HID, OUT), lambda i: (0, 0)),
            pl.BlockSpec((1, OUT), lambda i: (0, 0)),
        ],
        out_specs=pl.BlockSpec((B, OUT), lambda i: (0, 0)),
        out_shape=jax.ShapeDtypeStruct((B, OUT), jnp.float32),
        scratch_shapes=[
            pltpu.VMEM((B, HID), jnp.float32),
            pltpu.VMEM((B, HID), jnp.float32),
        ],
    )(num, den, batch3, gb, bnw, bnb, hwt, hb)


@jax.jit
def _run(graph_x, edge_index, batch, W_gat, att_src, att_dst, gat_bias,
         bn_weight, bn_bias, head_W, head_b):
    x = graph_x.astype(jnp.float32)

    A2 = jnp.zeros((HID, HID), jnp.float32)
    A2 = A2.at[:, 0].set(att_src).at[:, 1].set(att_dst)
    h, aux = _stage1(x, W_gat.T, A2)

    loop = jnp.arange(N, dtype=jnp.int32)
    src = jnp.concatenate(
        [edge_index[0], loop, jnp.zeros((EEP - EE,), jnp.int32)])
    dst = jnp.concatenate(
        [edge_index[1], loop, jnp.full((EEP - EE,), N, jnp.int32)])
    src3 = src.reshape(NW, NCHUNK, K)
    dst3 = dst.reshape(NW, NCHUNK, K)
    ed4 = jnp.stack([src3, dst3], axis=2)
    asrc_p = jnp.pad(aux[:, 0], (0, NAP - N))
    adst_p = jnp.pad(aux[:, 1], (0, NAP - N))

    num, den = _stage2(h, asrc_p, adst_p, ed4)

    bnw = (bn_weight / jnp.sqrt(1.0 + 1e-5)).reshape(1, HID)
    batch3 = jnp.pad(batch, (0, NP - N), constant_values=B).reshape(
        NP // 1024, 1, 1024)
    logits = _stage3(
        num, den, batch3, gat_bias.reshape(1, HID), bnw,
        bn_bias.reshape(1, HID), head_W.T, head_b.reshape(1, OUT))
    return logits


def kernel(graph_x, edge_index, batch, W_gat, att_src, att_dst, gat_bias,
           bn_weight, bn_bias, head_W, head_b):
    return _run(graph_x, edge_index, batch, W_gat, att_src, att_dst,
                gat_bias, bn_weight, bn_bias, head_W, head_b)


# speedup vs baseline: 37.0453x; 1.1230x over previous
import jax
import jax.numpy as jnp
from jax import lax
from jax.experimental import pallas as pl
from jax.experimental.pallas import tpu as pltpu
from jax.experimental.pallas import tpu_sc as plsc

N = 10000
E = 320000
D = 128
HID = 128
OUT = 16
B = 64

NC = 2
NS = 16
NW = NC * NS

EE = E + N
K = 48
T = 10320
NCHUNK = T // K
EEP = NW * T
NAP = N + 16
SLAB = 64
NP = 10240
ROWS_PER_TILE = NP // NS


def _vgather(x, idx):
    dnums = lax.GatherDimensionNumbers(
        offset_dims=(), collapsed_slice_dims=(0,), start_index_map=(0,))
    return lax.gather(
        x, idx[:, None], dimension_numbers=dnums, slice_sizes=(1,),
        mode=lax.GatherScatterMode.PROMISE_IN_BOUNDS)


def _mm_body(x_ref, wt_ref, a2_ref, h_ref, aux_ref):
    h = jnp.dot(x_ref[...], wt_ref[...], preferred_element_type=jnp.float32)
    h_ref[...] = h
    aux_ref[...] = jnp.dot(h, a2_ref[...], preferred_element_type=jnp.float32)


def _stage1(x, WT, A2):
    R = 1000
    grid = N // R
    return pl.pallas_call(
        _mm_body,
        grid=(grid,),
        in_specs=[
            pl.BlockSpec((R, D), lambda i: (i, 0)),
            pl.BlockSpec((D, HID), lambda i: (0, 0)),
            pl.BlockSpec((HID, HID), lambda i: (0, 0)),
        ],
        out_specs=[
            pl.BlockSpec((R, HID), lambda i: (i, 0)),
            pl.BlockSpec((R, HID), lambda i: (i, 0)),
        ],
        out_shape=[
            jax.ShapeDtypeStruct((N, HID), jnp.float32),
            jax.ShapeDtypeStruct((N, HID), jnp.float32),
        ],
    )(x, WT, A2)


def _sc_body(h_hbm, asrc_hbm, adst_hbm, ed_hbm,
             num_hbm, den_hbm,
             asrc_v, adst_v, ed_v, den_v, rows_v,
             num_sh, sem_e0, sem_e1, sem_e2, sem_g0, sem_g1, sem_s):
    cid = lax.axis_index("c")
    sid = lax.axis_index("s")
    wid = sid * NC + cid

    pltpu.sync_copy(asrc_hbm, asrc_v)
    pltpu.sync_copy(adst_hbm, adst_v)

    zero16 = jnp.zeros((16,), jnp.float32)

    def zero_rows(r, carry):
        for q in range(8):
            rows_v[r, pl.ds(q * 16, 16)] = zero16
        return carry

    lax.fori_loop(0, 2 * K, zero_rows, 0)

    def zero_den(c, carry):
        den_v[pl.ds(c * 16, 16)] = zero16
        return carry

    lax.fori_loop(0, NP // 16, zero_den, 0)

    base = sid * ROWS_PER_TILE

    def zero_num(c, carry):
        pltpu.sync_copy(rows_v.at[pl.ds(0, SLAB)],
                        num_sh.at[pl.ds(base + c * SLAB, SLAB)])
        return carry

    lax.fori_loop(0, ROWS_PER_TILE // SLAB, zero_num, 0)
    plsc.subcore_barrier()

    lanes = lax.broadcasted_iota(jnp.int32, (16,), 0)

    pltpu.async_copy(ed_hbm.at[wid, 0], ed_v.at[0], sem_e0)
    pltpu.async_copy(ed_hbm.at[wid, 1], ed_v.at[1], sem_e1)
    pltpu.make_async_copy(ed_hbm.at[wid, 0], ed_v.at[0], sem_e0).wait()
    pltpu.async_copy(
        h_hbm.at[ed_v.at[0, 0]], rows_v.at[pl.ds(0, K)], sem_g0)

    def chunk(j, carry):
        b = lax.rem(j, 2)
        nb = 1 - b
        boff = b * K
        sl = lax.rem(j, 3)
        sl1 = lax.rem(j + 1, 3)
        sl2 = lax.rem(j + 2, 3)
        sem_g_b = [sem_g0, sem_g1]
        sem_e_all = [sem_e0, sem_e1, sem_e2]

        @pl.when(j > 0)
        def _():
            pltpu.make_async_copy(
                rows_v.at[pl.ds(nb * K, K)],
                num_sh.at[ed_v.at[sl2, 1]], sem_s).wait()

        @pl.when(j + 1 < NCHUNK)
        def _():
            for s in range(3):
                @pl.when(sl1 == s)
                def _():
                    pltpu.make_async_copy(
                        ed_hbm.at[wid, j + 1], ed_v.at[sl1],
                        sem_e_all[s]).wait()
            for s in range(2):
                @pl.when(nb == s)
                def _():
                    pltpu.async_copy(
                        h_hbm.at[ed_v.at[sl1, 0]],
                        rows_v.at[pl.ds(nb * K, K)], sem_g_b[s])

        @pl.when(j + 2 < NCHUNK)
        def _():
            for s in range(3):
                @pl.when(sl2 == s)
                def _():
                    pltpu.async_copy(
                        ed_hbm.at[wid, j + 2], ed_v.at[sl2], sem_e_all[s])

        w16s = []
        for l in range(K // 16):
            s16 = ed_v[sl, 0, pl.ds(l * 16, 16)]
            d16 = ed_v[sl, 1, pl.ds(l * 16, 16)]
            a_s = plsc.load_gather(asrc_v, [s16])
            a_d = plsc.load_gather(adst_v, [d16])
            al = a_s + a_d
            al = jnp.where(al >= 0.0, al, al * 0.2)
            w = jnp.exp(al)
            w16s.append(w)
            k, v = plsc.sort_key_val(d16, w)
            cs = plsc.cumsum(v)
            prv = jnp.maximum(lanes - 1, 0)
            nxt = jnp.minimum(lanes + 1, 15)
            kprev = _vgather(k, prv)
            knext = _vgather(k, nxt)
            csprev = _vgather(cs, prv)
            is_first = (lanes == 0) | (k != kprev)
            is_last = (lanes == 15) | (k != knext)
            neg = jnp.where(lanes == 0, 0.0, -csprev)
            plsc.addupdate_scatter(den_v, [k], cs, mask=is_last)
            plsc.addupdate_scatter(den_v, [k], neg, mask=is_first)

        for s in range(2):
            @pl.when(b == s)
            def _():
                pltpu.make_async_copy(
                    h_hbm.at[ed_v.at[sl, 0]],
                    rows_v.at[pl.ds(boff, K)], sem_g_b[s]).wait()
        for l in range(K // 16):
            for e in range(16):
                wsp = _vgather(w16s[l], jnp.full((16,), e, jnp.int32))
                r = boff + l * 16 + e
                for q in range(8):
                    rows_v[r, pl.ds(q * 16, 16)] = (
                        rows_v[r, pl.ds(q * 16, 16)] * wsp)
        pltpu.async_copy(
            rows_v.at[pl.ds(boff, K)], num_sh.at[ed_v.at[sl, 1]],
            sem_s, add=True)
        return carry

    lax.fori_loop(0, NCHUNK, chunk, 0)
    lb = (NCHUNK - 1) % 2
    lsl = (NCHUNK - 1) % 3
    pltpu.make_async_copy(
        rows_v.at[pl.ds(lb * K, K)], num_sh.at[ed_v.at[lsl, 1]],
        sem_s).wait()
    plsc.subcore_barrier()

    def wb_chunk(c, carry):
        off = base + c * SLAB
        pltpu.sync_copy(num_sh.at[pl.ds(off, SLAB)],
                        rows_v.at[pl.ds(0, SLAB)])
        pltpu.sync_copy(rows_v.at[pl.ds(0, SLAB)],
                        num_hbm.at[cid, pl.ds(off, SLAB)])
        return carry

    lax.fori_loop(0, ROWS_PER_TILE // SLAB, wb_chunk, 0)
    pltpu.sync_copy(den_v, den_hbm.at[wid])


def _stage2(h, asrc_p, adst_p, ed4):
    mesh = plsc.VectorSubcoreMesh(core_axis_name="c", subcore_axis_name="s")
    run = pl.kernel(
        _sc_body,
        out_type=[
            jax.ShapeDtypeStruct((NC, NP, HID), jnp.float32),
            jax.ShapeDtypeStruct((NW, NP), jnp.float32),
        ],
        mesh=mesh,
        scratch_types=[
            pltpu.VMEM((NAP,), jnp.float32),
            pltpu.VMEM((NAP,), jnp.float32),
            pltpu.VMEM((3, 2, K), jnp.int32),
            pltpu.VMEM((NP,), jnp.float32),
            pltpu.VMEM((2 * K, HID), jnp.float32),
            pltpu.VMEM_SHARED((NP, HID), jnp.float32),
            pltpu.SemaphoreType.DMA,
            pltpu.SemaphoreType.DMA,
            pltpu.SemaphoreType.DMA,
            pltpu.SemaphoreType.DMA,
            pltpu.SemaphoreType.DMA,
            pltpu.SemaphoreType.DMA,
        ],
        compiler_params=pltpu.CompilerParams(needs_layout_passes=False),
    )
    return run(h, asrc_p, adst_p, ed4)


def _fin_body(num_ref, den_ref, batch_ref, gb_ref, bnw_ref, bnb_ref,
              hwt_ref, hb_ref, out_ref, acc_ref, cnt_ref):
    i = pl.program_id(0)
    ni = pl.num_programs(0)

    @pl.when(i == 0)
    def _():
        acc_ref[...] = jnp.zeros_like(acc_ref)
        cnt_ref[...] = jnp.zeros_like(cnt_ref)

    num = num_ref[0] + num_ref[1]
    den = jnp.sum(den_ref[...], axis=0)
    rec = 1.0 / (den + 1e-16)
    o = num * rec[:, None]
    o = o + gb_ref[...]
    o = o * bnw_ref[...] + bnb_ref[...]
    o = jnp.maximum(o, 0.0)

    b = batch_ref[0, 0, :]
    rows = b.shape[0]
    gids = lax.broadcasted_iota(jnp.int32, (B, rows), 0)
    m = (gids == b[None, :]).astype(jnp.float32)
    acc_ref[...] += jnp.dot(m, o, preferred_element_type=jnp.float32)
    cnt_ref[...] += jnp.broadcast_to(
        jnp.sum(m, axis=1)[:, None], cnt_ref.shape)

    @pl.when(i == ni - 1)
    def _():
        pooled = acc_ref[...] / jnp.maximum(cnt_ref[...], 1.0)
        out_ref[...] = (
            jnp.dot(pooled, hwt_ref[...], preferred_element_type=jnp.float32)
            + hb_ref[...])


def _stage3(num, den, batch3, gb, bnw, bnb, hwt, hb):
    R = 1024
    grid = NP // R
    return pl.pallas_call(
        _fin_body,
        grid=(grid,),
        in_specs=[
            pl.BlockSpec((NC, R, HID), lambda i: (0, i, 0)),
            pl.BlockSpec((NW, R), lambda i: (0, i)),
            pl.BlockSpec((1, 1, R), lambda i: (i, 0, 0)),
            pl.BlockSpec((1, HID), lambda i: (0, 0)),
            pl.BlockSpec((1, HID), lambda i: (0, 0)),
            pl.BlockSpec((1, HID), lambda i: (0, 0)),
            pl.BlockSpec((HID, OUT), lambda i: (0, 0)),
            pl.BlockSpec((1, OUT), lambda i: (0, 0)),
        ],
        out_specs=pl.BlockSpec((B, OUT), lambda i: (0, 0)),
        out_shape=jax.ShapeDtypeStruct((B, OUT), jnp.float32),
        scratch_shapes=[
            pltpu.VMEM((B, HID), jnp.float32),
            pltpu.VMEM((B, HID), jnp.float32),
        ],
    )(num, den, batch3, gb, bnw, bnb, hwt, hb)


@jax.jit
def _run(graph_x, edge_index, batch, W_gat, att_src, att_dst, gat_bias,
         bn_weight, bn_bias, head_W, head_b):
    x = graph_x.astype(jnp.float32)

    A2 = jnp.zeros((HID, HID), jnp.float32)
    A2 = A2.at[:, 0].set(att_src).at[:, 1].set(att_dst)
    h, aux = _stage1(x, W_gat.T, A2)

    loop = jnp.arange(N, dtype=jnp.int32)
    src = jnp.concatenate(
        [edge_index[0], loop, jnp.zeros((EEP - EE,), jnp.int32)])
    dst = jnp.concatenate(
        [edge_index[1], loop, jnp.full((EEP - EE,), N, jnp.int32)])
    src3 = src.reshape(NW, NCHUNK, K)
    dst3 = dst.reshape(NW, NCHUNK, K)
    ed4 = jnp.stack([src3, dst3], axis=2)
    asrc_p = jnp.pad(aux[:, 0], (0, NAP - N))
    adst_p = jnp.pad(aux[:, 1], (0, NAP - N))

    num, den = _stage2(h, asrc_p, adst_p, ed4)

    bnw = (bn_weight / jnp.sqrt(1.0 + 1e-5)).reshape(1, HID)
    batch3 = jnp.pad(batch, (0, NP - N), constant_values=B).reshape(
        NP // 1024, 1, 1024)
    logits = _stage3(
        num, den, batch3, gat_bias.reshape(1, HID), bnw,
        bn_bias.reshape(1, HID), head_W.T, head_b.reshape(1, OUT))
    return logits


def kernel(graph_x, edge_index, batch, W_gat, att_src, att_dst, gat_bias,
           bn_weight, bn_bias, head_W, head_b):
    return _run(graph_x, edge_index, batch, W_gat, att_src, att_dst,
                gat_bias, bn_weight, bn_bias, head_W, head_b)


# speedup vs baseline: 37.5740x; 1.0143x over previous
import jax
import jax.numpy as jnp
from jax import lax
from jax.experimental import pallas as pl
from jax.experimental.pallas import tpu as pltpu
from jax.experimental.pallas import tpu_sc as plsc

N = 10000
E = 320000
D = 128
HID = 128
OUT = 16
B = 64

NC = 2
NS = 16
NW = NC * NS

EE = E + N
K = 48
T = 10320
NCHUNK = T // K
EEP = NW * T
NAP = N + 16
SLAB = 64
NP = 10240
ROWS_PER_TILE = NP // NS


def _vgather(x, idx):
    dnums = lax.GatherDimensionNumbers(
        offset_dims=(), collapsed_slice_dims=(0,), start_index_map=(0,))
    return lax.gather(
        x, idx[:, None], dimension_numbers=dnums, slice_sizes=(1,),
        mode=lax.GatherScatterMode.PROMISE_IN_BOUNDS)


def _mm_body(x_ref, wt_ref, a2_ref, h_ref, aux_ref):
    h = jnp.dot(x_ref[...], wt_ref[...], preferred_element_type=jnp.float32)
    h_ref[...] = h
    aux_ref[...] = jnp.dot(h, a2_ref[...], preferred_element_type=jnp.float32)


def _stage1(x, WT, A2):
    R = 1000
    grid = N // R
    return pl.pallas_call(
        _mm_body,
        grid=(grid,),
        in_specs=[
            pl.BlockSpec((R, D), lambda i: (i, 0)),
            pl.BlockSpec((D, HID), lambda i: (0, 0)),
            pl.BlockSpec((HID, HID), lambda i: (0, 0)),
        ],
        out_specs=[
            pl.BlockSpec((R, HID), lambda i: (i, 0)),
            pl.BlockSpec((R, HID), lambda i: (i, 0)),
        ],
        out_shape=[
            jax.ShapeDtypeStruct((N, HID), jnp.float32),
            jax.ShapeDtypeStruct((N, HID), jnp.float32),
        ],
    )(x, WT, A2)


def _sc_body(h_hbm, asrc_hbm, adst_hbm, ed_hbm,
             num_hbm, den_hbm,
             asrc_v, adst_v, ed_v, den_v, rows_v,
             num_sh, sem_e0, sem_e1, sem_e2, sem_g0, sem_g1, sem_s):
    cid = lax.axis_index("c")
    sid = lax.axis_index("s")
    wid = sid * NC + cid

    pltpu.async_copy(asrc_hbm, asrc_v, sem_g0)
    pltpu.async_copy(adst_hbm, adst_v, sem_g1)

    zero16 = jnp.zeros((16,), jnp.float32)

    def zero_rows(r, carry):
        for q in range(8):
            rows_v[r, pl.ds(q * 16, 16)] = zero16
        return carry

    lax.fori_loop(0, 2 * K, zero_rows, 0)

    def zero_den(c, carry):
        den_v[pl.ds(c * 16, 16)] = zero16
        return carry

    lax.fori_loop(0, NP // 16, zero_den, 0)

    base = sid * ROWS_PER_TILE

    def zero_num(c, carry):
        pltpu.sync_copy(rows_v.at[pl.ds(0, SLAB)],
                        num_sh.at[pl.ds(base + c * SLAB, SLAB)])
        return carry

    lax.fori_loop(0, ROWS_PER_TILE // SLAB, zero_num, 0)
    pltpu.make_async_copy(asrc_hbm, asrc_v, sem_g0).wait()
    pltpu.make_async_copy(adst_hbm, adst_v, sem_g1).wait()
    plsc.subcore_barrier()

    lanes = lax.broadcasted_iota(jnp.int32, (16,), 0)

    pltpu.async_copy(ed_hbm.at[wid, 0], ed_v.at[0], sem_e0)
    pltpu.async_copy(ed_hbm.at[wid, 1], ed_v.at[1], sem_e1)
    pltpu.make_async_copy(ed_hbm.at[wid, 0], ed_v.at[0], sem_e0).wait()
    pltpu.async_copy(
        h_hbm.at[ed_v.at[0, 0]], rows_v.at[pl.ds(0, K)], sem_g0)

    def chunk(j, carry):
        b = lax.rem(j, 2)
        nb = 1 - b
        boff = b * K
        sl = lax.rem(j, 3)
        sl1 = lax.rem(j + 1, 3)
        sl2 = lax.rem(j + 2, 3)
        sem_g_b = [sem_g0, sem_g1]
        sem_e_all = [sem_e0, sem_e1, sem_e2]

        @pl.when(j > 0)
        def _():
            pltpu.make_async_copy(
                rows_v.at[pl.ds(nb * K, K)],
                num_sh.at[ed_v.at[sl2, 1]], sem_s).wait()

        @pl.when(j + 1 < NCHUNK)
        def _():
            for s in range(3):
                @pl.when(sl1 == s)
                def _():
                    pltpu.make_async_copy(
                        ed_hbm.at[wid, j + 1], ed_v.at[sl1],
                        sem_e_all[s]).wait()
            for s in range(2):
                @pl.when(nb == s)
                def _():
                    pltpu.async_copy(
                        h_hbm.at[ed_v.at[sl1, 0]],
                        rows_v.at[pl.ds(nb * K, K)], sem_g_b[s])

        @pl.when(j + 2 < NCHUNK)
        def _():
            for s in range(3):
                @pl.when(sl2 == s)
                def _():
                    pltpu.async_copy(
                        ed_hbm.at[wid, j + 2], ed_v.at[sl2], sem_e_all[s])

        w16s = []
        for l in range(K // 16):
            s16 = ed_v[sl, 0, pl.ds(l * 16, 16)]
            d16 = ed_v[sl, 1, pl.ds(l * 16, 16)]
            a_s = plsc.load_gather(asrc_v, [s16])
            a_d = plsc.load_gather(adst_v, [d16])
            al = a_s + a_d
            al = jnp.where(al >= 0.0, al, al * 0.2)
            w = jnp.exp(al)
            w16s.append(w)
            k, v = plsc.sort_key_val(d16, w)
            cs = plsc.cumsum(v)
            prv = jnp.maximum(lanes - 1, 0)
            nxt = jnp.minimum(lanes + 1, 15)
            kprev = _vgather(k, prv)
            knext = _vgather(k, nxt)
            csprev = _vgather(cs, prv)
            is_first = (lanes == 0) | (k != kprev)
            is_last = (lanes == 15) | (k != knext)
            neg = jnp.where(lanes == 0, 0.0, -csprev)
            plsc.addupdate_scatter(den_v, [k], cs, mask=is_last)
            plsc.addupdate_scatter(den_v, [k], neg, mask=is_first)

        for s in range(2):
            @pl.when(b == s)
            def _():
                pltpu.make_async_copy(
                    h_hbm.at[ed_v.at[sl, 0]],
                    rows_v.at[pl.ds(boff, K)], sem_g_b[s]).wait()
        for l in range(K // 16):
            for e in range(16):
                wsp = _vgather(w16s[l], jnp.full((16,), e, jnp.int32))
                r = boff + l * 16 + e
                for q in range(8):
                    rows_v[r, pl.ds(q * 16, 16)] = (
                        rows_v[r, pl.ds(q * 16, 16)] * wsp)
        pltpu.async_copy(
            rows_v.at[pl.ds(boff, K)], num_sh.at[ed_v.at[sl, 1]],
            sem_s, add=True)
        return carry

    lax.fori_loop(0, NCHUNK, chunk, 0)
    lb = (NCHUNK - 1) % 2
    lsl = (NCHUNK - 1) % 3
    pltpu.make_async_copy(
        rows_v.at[pl.ds(lb * K, K)], num_sh.at[ed_v.at[lsl, 1]],
        sem_s).wait()
    plsc.subcore_barrier()

    def wb_chunk(c, carry):
        off = base + c * SLAB
        pltpu.sync_copy(num_sh.at[pl.ds(off, SLAB)],
                        num_hbm.at[cid, pl.ds(off, SLAB)])
        return carry

    lax.fori_loop(0, ROWS_PER_TILE // SLAB, wb_chunk, 0)
    pltpu.sync_copy(den_v, den_hbm.at[wid])


def _stage2(h, asrc_p, adst_p, ed4):
    mesh = plsc.VectorSubcoreMesh(core_axis_name="c", subcore_axis_name="s")
    run = pl.kernel(
        _sc_body,
        out_type=[
            jax.ShapeDtypeStruct((NC, NP, HID), jnp.float32),
            jax.ShapeDtypeStruct((NW, NP), jnp.float32),
        ],
        mesh=mesh,
        scratch_types=[
            pltpu.VMEM((NAP,), jnp.float32),
            pltpu.VMEM((NAP,), jnp.float32),
            pltpu.VMEM((3, 2, K), jnp.int32),
            pltpu.VMEM((NP,), jnp.float32),
            pltpu.VMEM((2 * K, HID), jnp.float32),
            pltpu.VMEM_SHARED((NP, HID), jnp.float32),
            pltpu.SemaphoreType.DMA,
            pltpu.SemaphoreType.DMA,
            pltpu.SemaphoreType.DMA,
            pltpu.SemaphoreType.DMA,
            pltpu.SemaphoreType.DMA,
            pltpu.SemaphoreType.DMA,
        ],
        compiler_params=pltpu.CompilerParams(needs_layout_passes=False),
    )
    return run(h, asrc_p, adst_p, ed4)


def _fin_body(num_ref, den_ref, batch_ref, gb_ref, bnw_ref, bnb_ref,
              hwt_ref, hb_ref, out_ref, acc_ref, cnt_ref):
    i = pl.program_id(0)
    ni = pl.num_programs(0)

    @pl.when(i == 0)
    def _():
        acc_ref[...] = jnp.zeros_like(acc_ref)
        cnt_ref[...] = jnp.zeros_like(cnt_ref)

    num = num_ref[0] + num_ref[1]
    den = jnp.sum(den_ref[...], axis=0)
    rec = 1.0 / (den + 1e-16)
    o = num * rec[:, None]
    o = o + gb_ref[...]
    o = o * bnw_ref[...] + bnb_ref[...]
    o = jnp.maximum(o, 0.0)

    b = batch_ref[0, 0, :]
    rows = b.shape[0]
    gids = lax.broadcasted_iota(jnp.int32, (B, rows), 0)
    m = (gids == b[None, :]).astype(jnp.float32)
    acc_ref[...] += jnp.dot(m, o, preferred_element_type=jnp.float32)
    cnt_ref[...] += jnp.broadcast_to(
        jnp.sum(m, axis=1)[:, None], cnt_ref.shape)

    @pl.when(i == ni - 1)
    def _():
        pooled = acc_ref[...] / jnp.maximum(cnt_ref[...], 1.0)
        out_ref[...] = (
            jnp.dot(pooled, hwt_ref[...], preferred_element_type=jnp.float32)
            + hb_ref[...])


def _stage3(num, den, batch3, gb, bnw, bnb, hwt, hb):
    R = 1024
    grid = NP // R
    return pl.pallas_call(
        _fin_body,
        grid=(grid,),
        in_specs=[
            pl.BlockSpec((NC, R, HID), lambda i: (0, i, 0)),
            pl.BlockSpec((NW, R), lambda i: (0, i)),
            pl.BlockSpec((1, 1, R), lambda i: (i, 0, 0)),
            pl.BlockSpec((1, HID), lambda i: (0, 0)),
            pl.BlockSpec((1, HID), lambda i: (0, 0)),
            pl.BlockSpec((1, HID), lambda i: (0, 0)),
            pl.BlockSpec((HID, OUT), lambda i: (0, 0)),
            pl.BlockSpec((1, OUT), lambda i: (0, 0)),
        ],
        out_specs=pl.BlockSpec((B, OUT), lambda i: (0, 0)),
        out_shape=jax.ShapeDtypeStruct((B, OUT), jnp.float32),
        scratch_shapes=[
            pltpu.VMEM((B, HID), jnp.float32),
            pltpu.VMEM((B, HID), jnp.float32),
        ],
    )(num, den, batch3, gb, bnw, bnb, hwt, hb)


@jax.jit
def _run(graph_x, edge_index, batch, W_gat, att_src, att_dst, gat_bias,
         bn_weight, bn_bias, head_W, head_b):
    x = graph_x.astype(jnp.float32)

    A2 = jnp.zeros((HID, HID), jnp.float32)
    A2 = A2.at[:, 0].set(att_src).at[:, 1].set(att_dst)
    h, aux = _stage1(x, W_gat.T, A2)

    loop = jnp.arange(N, dtype=jnp.int32)
    src = jnp.concatenate(
        [edge_index[0], loop, jnp.zeros((EEP - EE,), jnp.int32)])
    dst = jnp.concatenate(
        [edge_index[1], loop, jnp.full((EEP - EE,), N, jnp.int32)])
    src3 = src.reshape(NW, NCHUNK, K)
    dst3 = dst.reshape(NW, NCHUNK, K)
    ed4 = jnp.stack([src3, dst3], axis=2)
    asrc_p = jnp.pad(aux[:, 0], (0, NAP - N))
    adst_p = jnp.pad(aux[:, 1], (0, NAP - N))

    num, den = _stage2(h, asrc_p, adst_p, ed4)

    bnw = (bn_weight / jnp.sqrt(1.0 + 1e-5)).reshape(1, HID)
    batch3 = jnp.pad(batch, (0, NP - N), constant_values=B).reshape(
        NP // 1024, 1, 1024)
    logits = _stage3(
        num, den, batch3, gat_bias.reshape(1, HID), bnw,
        bn_bias.reshape(1, HID), head_W.T, head_b.reshape(1, OUT))
    return logits


def kernel(graph_x, edge_index, batch, W_gat, att_src, att_dst, gat_bias,
           bn_weight, bn_bias, head_W, head_b):
    return _run(graph_x, edge_index, batch, W_gat, att_src, att_dst,
                gat_bias, bn_weight, bn_bias, head_W, head_b)
